# baseline (speedup 1.0000x reference)
import functools
import math

import jax
import jax.numpy as jnp
from jax import lax
from jax.experimental import pallas as pl
from jax.experimental.pallas import tpu as pltpu
from jax.experimental.pallas import tpu_sc as plsc

_INTERPRET = False
_NC = 16
_HIGH = jax.lax.Precision.HIGHEST


def _round_up(a, b):
    return (a + b - 1) // b * b


def _dot(a, b):
    return jax.lax.dot_general(a, b, (((1,), (0,)), ((), ())),
                               preferred_element_type=jnp.float32)


def _bf(x):
    return x.astype(jnp.bfloat16).astype(jnp.float32)


def _knn(points, queries, K):
    B, N, _ = points.shape
    M = queries.shape[1]
    TM = min(M, 256)
    ptT = jnp.swapaxes(points, 1, 2)

    def body(pt_ref, q_ref, o_ref):
        b = pl.program_id(0)
        pt = pt_ref[0]
        q = q_ref[0]
        qp = _dot(q, pt)
        psq = jnp.sum(pt * pt, axis=0, keepdims=True)
        qsq = jnp.sum(q * q, axis=1, keepdims=True)
        d2 = qsq - 2.0 * qp + psq
        iota = jax.lax.broadcasted_iota(jnp.int32, (TM, N), 1) + b * N
        big_i = jnp.int32(2**31 - 1)
        cols = []
        for _ in range(K):
            vmin = jnp.min(d2, axis=1, keepdims=True)
            eq = d2 == vmin
            sel = jnp.min(jnp.where(eq, iota, big_i), axis=1, keepdims=True)
            cols.append(sel)
            d2 = jnp.where(iota == sel, jnp.float32(jnp.inf), d2)
        o_ref[0] = jnp.concatenate(cols, axis=1)

    return pl.pallas_call(
        body,
        grid=(B, M // TM),
        in_specs=[
            pl.BlockSpec((1, 3, N), lambda b, i: (b, 0, 0)),
            pl.BlockSpec((1, TM, 3), lambda b, i: (b, i, 0)),
        ],
        out_specs=pl.BlockSpec((1, TM, K), lambda b, i: (b, i, 0)),
        out_shape=jax.ShapeDtypeStruct((B, M, K), jnp.int32),
        interpret=_INTERPRET,
    )(ptT, queries)


@functools.lru_cache(maxsize=None)
def _make_sc_gather(T, D, R):
    nchunks = R // 128
    NW = 32
    c_per_w = -(-nchunks // NW)
    mesh = plsc.VectorSubcoreMesh(core_axis_name="c", subcore_axis_name="s")

    @functools.partial(
        pl.kernel, mesh=mesh,
        out_type=jax.ShapeDtypeStruct((R, D), jnp.float32),
        scratch_types=[
            pltpu.VMEM((128,), jnp.int32),
            pltpu.VMEM((128, D), jnp.float32),
            pltpu.SemaphoreType.DMA,
        ],
    )
    def k(table_hbm, idx_hbm, out_hbm, idx_v, rows_v, sem):
        wid = lax.axis_index("s") * 2 + lax.axis_index("c")

        def body(j, carry):
            chunk = wid + NW * j

            @pl.when(chunk < nchunks)
            def _():
                pltpu.sync_copy(idx_hbm.at[chunk], idx_v)
                pltpu.async_copy(table_hbm.at[idx_v], rows_v, sem).wait()
                pltpu.sync_copy(rows_v, out_hbm.at[pl.ds(chunk * 128, 128)])
            return carry

        lax.fori_loop(0, c_per_w, body, 0)

    return k


def _sc_gather(table, idx2d):
    T, D = table.shape
    R = idx2d.shape[0] * 128
    return _make_sc_gather(T, D, R)(table, idx2d)


def _compute(g, q, prm, K, C, D):
    rows = q.shape[0]
    nc = _NC
    JC = nc * C
    TM = min(rows, max(8, 1 << (3_000_000 // (K * JC)).bit_length() - 1),
             256)
    RK = TM * K
    cout = prm['weight'].shape[2]

    cen = jnp.pad(prm['centers'], ((0, 5), (0, 0)))
    l1b = jnp.broadcast_to(prm['l1b'][None, :], (8, 2 * nc))
    l2b = jnp.broadcast_to(prm['l2b'][None, :], (8, nc))
    l3b = jnp.broadcast_to(prm['l3b'][None, :], (8, nc))
    r1m = jnp.repeat(jnp.eye(nc, dtype=jnp.float32), C, axis=1)
    r2m = jnp.tile(jnp.eye(C, dtype=jnp.float32), (1, nc))
    w2p = jnp.swapaxes(prm['weight'], 0, 1).reshape(JC, cout)

    def body(g_ref, q_ref, cen_ref, w1_ref, b1_ref, w2_ref, b2_ref,
             w3_ref, b3_ref, r1_ref, r2_ref, w2p_ref, o_ref):
        gg = g_ref[...]
        rel = gg[:, 0:3] - q_ref[...]
        blocks = [rel[:, d:d + 1] - cen_ref[d:d + 1, :] for d in range(3)]
        dmat = jnp.concatenate(blocks, axis=1)
        h = jnp.maximum(_dot(dmat, w1_ref[...]) + b1_ref[0:1, :], 0.0)
        h = jnp.maximum(_dot(h, w2_ref[...]) + b2_ref[0:1, :], 0.0)
        h = jnp.maximum(_dot(h, w3_ref[...]) + b3_ref[0:1, :], 0.0)
        fg = gg[:, 3:3 + C]
        h2 = _dot(_bf(h), r1_ref[...])
        fg2 = _dot(_bf(fg), r2_ref[...])
        f2 = jnp.sum((h2 * fg2).reshape(TM, K, JC), axis=1)
        o_ref[...] = _dot(_bf(f2), w2p_ref[...]) * (1.0 / K)

    full = lambda a: pl.BlockSpec(a.shape, lambda i: (0,) * a.ndim)
    return pl.pallas_call(
        body,
        grid=(rows // TM,),
        in_specs=[
            pl.BlockSpec((RK, D), lambda i: (i, 0)),
            pl.BlockSpec((RK, 3), lambda i: (i, 0)),
            full(cen), full(prm['l1w']), full(l1b),
            full(prm['l2w']), full(l2b),
            full(prm['l3w']), full(l3b), full(r1m), full(r2m), full(w2p),
        ],
        out_specs=pl.BlockSpec((TM, cout), lambda i: (i, 0)),
        out_shape=jax.ShapeDtypeStruct((rows, cout), jnp.float32),
        interpret=_INTERPRET,
    )(g, jnp.repeat(q, K, axis=0), cen, prm['l1w'], l1b, prm['l2w'], l2b,
      prm['l3w'], l3b, r1m, r2m, w2p)


def _compute_fused(idx, queries, table, prm, K, C):
    B, M, _ = queries.shape
    N = table.shape[1]
    CC = 3 + C
    nc = _NC
    JC = nc * C
    TM = min(M, 256 if N <= 256 else 128,
             max(8, 1 << (3_000_000 // (K * JC)).bit_length() - 1))
    RK = TM * K
    cout = prm['weight'].shape[2]

    cen = jnp.pad(prm['centers'], ((0, 5), (0, 0)))
    l1b = jnp.broadcast_to(prm['l1b'][None, :], (8, 2 * nc))
    l2b = jnp.broadcast_to(prm['l2b'][None, :], (8, nc))
    l3b = jnp.broadcast_to(prm['l3b'][None, :], (8, nc))
    r1m = jnp.repeat(jnp.eye(nc, dtype=jnp.float32), C, axis=1)
    r2m = jnp.tile(jnp.eye(C, dtype=jnp.float32), (1, nc))
    w2p = jnp.swapaxes(prm['weight'], 0, 1).reshape(JC, cout)

    def body(idx_ref, q_ref, t_ref, cen_ref, w1_ref, b1_ref, w2_ref, b2_ref,
             w3_ref, b3_ref, r1_ref, r2_ref, w2p_ref, o_ref):
        b = pl.program_id(0)
        idxl = idx_ref[0] - b * N
        tb = t_ref[0]
        ii = jax.lax.broadcasted_iota(jnp.int32, (RK, N), 1)
        oh = jnp.where(ii == idxl, 1.0, 0.0)
        coords = jax.lax.dot_general(oh, tb[:, 0:3], (((1,), (0,)), ((), ())),
                                     precision=_HIGH,
                                     preferred_element_type=jnp.float32)
        feats = _dot(oh, tb[:, 3:CC])
        rel = coords - q_ref[0]
        blocks = [rel[:, d:d + 1] - cen_ref[d:d + 1, :] for d in range(3)]
        dmat = jnp.concatenate(blocks, axis=1)
        h = jnp.maximum(_dot(dmat, w1_ref[...]) + b1_ref[0:1, :], 0.0)
        h = jnp.maximum(_dot(h, w2_ref[...]) + b2_ref[0:1, :], 0.0)
        h = jnp.maximum(_dot(h, w3_ref[...]) + b3_ref[0:1, :], 0.0)
        h2 = _dot(_bf(h), r1_ref[...])
        fg2 = _dot(_bf(feats), r2_ref[...])
        f2 = jnp.sum((h2 * fg2).reshape(TM, K, JC), axis=1)
        o_ref[0] = _dot(_bf(f2), w2p_ref[...]) * (1.0 / K)

    full = lambda a: pl.BlockSpec(a.shape, lambda b, i: (0,) * a.ndim)
    return pl.pallas_call(
        body,
        grid=(B, M // TM),
        in_specs=[
            pl.BlockSpec((1, RK, 1), lambda b, i: (b, i, 0)),
            pl.BlockSpec((1, RK, 3), lambda b, i: (b, i, 0)),
            pl.BlockSpec((1, N, CC), lambda b, i: (b, 0, 0)),
            full(cen), full(prm['l1w']), full(l1b),
            full(prm['l2w']), full(l2b),
            full(prm['l3w']), full(l3b), full(r1m), full(r2m), full(w2p),
        ],
        out_specs=pl.BlockSpec((1, TM, cout), lambda b, i: (b, i, 0)),
        out_shape=jax.ShapeDtypeStruct((B, M, cout), jnp.float32),
        interpret=_INTERPRET,
    )(idx.reshape(B, M * K, 1), jnp.repeat(queries, K, axis=1), table,
      cen, prm['l1w'], l1b, prm['l2w'], l2b, prm['l3w'], l3b, r1m, r2m, w2p)


def _bn_relu(x):
    B, M, C = x.shape
    x2 = x.reshape(B * M, C)

    def body(x_ref, o_ref):
        v = x_ref[...]
        mean = jnp.mean(v, axis=0, keepdims=True)
        xc = v - mean
        var = jnp.mean(xc * xc, axis=0, keepdims=True)
        o_ref[...] = jnp.maximum(xc / jnp.sqrt(var + 1e-5), 0.0)

    out = pl.pallas_call(
        body,
        out_shape=jax.ShapeDtypeStruct(x2.shape, jnp.float32),
        interpret=_INTERPRET,
    )(x2)
    return out.reshape(B, M, C)


def _classifier(x, w, b):
    B, M, C = x.shape
    x2 = x.reshape(B * M, C)
    cout = w.shape[1]
    b2 = jnp.broadcast_to(b[None, :], (8, cout))
    TM = min(B * M, 8192)

    def body(x_ref, w_ref, b_ref, o_ref):
        o_ref[...] = _dot(x_ref[...], w_ref[...]) + b_ref[0:1, :]

    out = pl.pallas_call(
        body,
        grid=((B * M) // TM,),
        in_specs=[
            pl.BlockSpec((TM, C), lambda i: (i, 0)),
            pl.BlockSpec(w.shape, lambda i: (0, 0)),
            pl.BlockSpec(b2.shape, lambda i: (0, 0)),
        ],
        out_specs=pl.BlockSpec((TM, cout), lambda i: (i, 0)),
        out_shape=jax.ShapeDtypeStruct((B * M, cout), jnp.float32),
        interpret=_INTERPRET,
    )(x2, w, b2)
    return out.reshape(B, M, cout)


def _ptconv(prm, feats, points, K, next_pts):
    B, N, C = feats.shape
    if isinstance(next_pts, int):
        if points.shape[1] != next_pts:
            stride = points.shape[1] // next_pts
            queries = points[:, ::stride, :][:, :next_pts, :]
        else:
            queries = points
    else:
        queries = next_pts
    M = queries.shape[1]

    idx = _knn(points, queries, K)
    if N <= 1024 and B * M * K * N <= 100_000_000:
        table3 = jnp.concatenate([points, feats], axis=2)
        out = _compute_fused(idx, queries, table3, prm, K, C)
        return out, queries
    D = _round_up(3 + C, 128)
    table = jnp.concatenate([points, feats], axis=2).reshape(B * N, 3 + C)
    if D > 3 + C:
        table = jnp.pad(table, ((0, 0), (0, D - (3 + C))))
    g = _sc_gather(table, idx.reshape(-1, 128))
    out = _compute(g, queries.reshape(B * M, 3), prm, K, C, D)
    return out.reshape(B, M, -1), queries


def kernel(x, input_pts, params):
    P = params
    x2, pts2 = _ptconv(P['cv2'], x, input_pts, 16, 1024)
    x2 = _bn_relu(x2)
    x3, pts3 = _ptconv(P['cv3'], x2, pts2, 16, 256)
    x3 = _bn_relu(x3)
    x4, pts4 = _ptconv(P['cv4'], x3, pts3, 8, 64)
    x4 = _bn_relu(x4)
    x5, pts5 = _ptconv(P['cv5'], x4, pts4, 8, 16)
    x5 = _bn_relu(x5)
    x6, pts6 = _ptconv(P['cv6'], x5, pts5, 4, 8)
    x6 = _bn_relu(x6)
    x5d, _ = _ptconv(P['cv5d'], x6, pts6, 4, pts5)
    x5d = _bn_relu(x5d)
    x5d = jnp.concatenate([x5d, x5], axis=2)
    x4d, _ = _ptconv(P['cv4d'], x5d, pts5, 4, pts4)
    x4d = _bn_relu(x4d)
    x4d = jnp.concatenate([x4d, x4], axis=2)
    x3d, _ = _ptconv(P['cv3d'], x4d, pts4, 4, pts3)
    x3d = _bn_relu(x3d)
    x3d = jnp.concatenate([x3d, x3], axis=2)
    x2d, _ = _ptconv(P['cv2d'], x3d, pts3, 8, pts2)
    x2d = _bn_relu(x2d)
    x2d = jnp.concatenate([x2d, x2], axis=2)
    x1d, _ = _ptconv(P['cv1d'], x2d, pts2, 8, input_pts)
    x1d = _bn_relu(x1d)
    return _classifier(x1d, P['fcout_w'], P['fcout_b'])


# speedup vs baseline: 1.0243x; 1.0243x over previous
import functools
import math

import jax
import jax.numpy as jnp
from jax import lax
from jax.experimental import pallas as pl
from jax.experimental.pallas import tpu as pltpu
from jax.experimental.pallas import tpu_sc as plsc

_INTERPRET = False
_NC = 16
_HIGH = jax.lax.Precision.HIGHEST


def _round_up(a, b):
    return (a + b - 1) // b * b


def _dot(a, b):
    return jax.lax.dot_general(a, b, (((1,), (0,)), ((), ())),
                               preferred_element_type=jnp.float32)


def _bf(x):
    return x.astype(jnp.bfloat16).astype(jnp.float32)


def _knn(points, queries, K):
    B, N, _ = points.shape
    M = queries.shape[1]
    TM = min(M, 512)
    ptT = jnp.swapaxes(points, 1, 2)

    def body(pt_ref, q_ref, o_ref):
        b = pl.program_id(0)
        pt = pt_ref[0]
        q = q_ref[0]
        qp = _dot(q, pt)
        psq = jnp.sum(pt * pt, axis=0, keepdims=True)
        qsq = jnp.sum(q * q, axis=1, keepdims=True)
        d2 = qsq - 2.0 * qp + psq
        iota = jax.lax.broadcasted_iota(jnp.int32, (TM, N), 1) + b * N
        big_i = jnp.int32(2**31 - 1)
        cols = []
        for _ in range(K):
            vmin = jnp.min(d2, axis=1, keepdims=True)
            eq = d2 == vmin
            sel = jnp.min(jnp.where(eq, iota, big_i), axis=1, keepdims=True)
            cols.append(sel)
            d2 = jnp.where(iota == sel, jnp.float32(jnp.inf), d2)
        o_ref[0] = jnp.concatenate(cols, axis=1)

    return pl.pallas_call(
        body,
        grid=(B, M // TM),
        in_specs=[
            pl.BlockSpec((1, 3, N), lambda b, i: (b, 0, 0)),
            pl.BlockSpec((1, TM, 3), lambda b, i: (b, i, 0)),
        ],
        out_specs=pl.BlockSpec((1, TM, K), lambda b, i: (b, i, 0)),
        out_shape=jax.ShapeDtypeStruct((B, M, K), jnp.int32),
        interpret=_INTERPRET,
    )(ptT, queries)


@functools.lru_cache(maxsize=None)
def _make_sc_gather(T, D, R):
    nchunks = R // 128
    NW = 32
    c_per_w = -(-nchunks // NW)
    mesh = plsc.VectorSubcoreMesh(core_axis_name="c", subcore_axis_name="s")

    @functools.partial(
        pl.kernel, mesh=mesh,
        out_type=jax.ShapeDtypeStruct((R, D), jnp.float32),
        scratch_types=[
            pltpu.VMEM((128,), jnp.int32),
            pltpu.VMEM((128, D), jnp.float32),
            pltpu.SemaphoreType.DMA,
        ],
    )
    def k(table_hbm, idx_hbm, out_hbm, idx_v, rows_v, sem):
        wid = lax.axis_index("s") * 2 + lax.axis_index("c")

        def body(j, carry):
            chunk = wid + NW * j

            @pl.when(chunk < nchunks)
            def _():
                pltpu.sync_copy(idx_hbm.at[chunk], idx_v)
                pltpu.async_copy(table_hbm.at[idx_v], rows_v, sem).wait()
                pltpu.sync_copy(rows_v, out_hbm.at[pl.ds(chunk * 128, 128)])
            return carry

        lax.fori_loop(0, c_per_w, body, 0)

    return k


def _sc_gather(table, idx2d):
    T, D = table.shape
    R = idx2d.shape[0] * 128
    return _make_sc_gather(T, D, R)(table, idx2d)


def _compute(g, q, prm, K, C, D):
    rows = q.shape[0]
    nc = _NC
    JC = nc * C
    TM = min(rows, max(8, 1 << (3_000_000 // (K * JC)).bit_length() - 1),
             256)
    RK = TM * K
    cout = prm['weight'].shape[2]

    cen = jnp.pad(prm['centers'], ((0, 5), (0, 0)))
    l1b = jnp.broadcast_to(prm['l1b'][None, :], (8, 2 * nc))
    l2b = jnp.broadcast_to(prm['l2b'][None, :], (8, nc))
    l3b = jnp.broadcast_to(prm['l3b'][None, :], (8, nc))
    r1m = jnp.repeat(jnp.eye(nc, dtype=jnp.float32), C, axis=1)
    r2m = jnp.tile(jnp.eye(C, dtype=jnp.float32), (1, nc))
    w2p = jnp.swapaxes(prm['weight'], 0, 1).reshape(JC, cout)

    def body(g_ref, q_ref, cen_ref, w1_ref, b1_ref, w2_ref, b2_ref,
             w3_ref, b3_ref, r1_ref, r2_ref, w2p_ref, o_ref):
        gg = g_ref[...]
        qq = q_ref[...]
        qrep = jnp.broadcast_to(qq[:, None, :], (TM, K, 3)).reshape(RK, 3)
        rel = gg[:, 0:3] - qrep
        blocks = [rel[:, d:d + 1] - cen_ref[d:d + 1, :] for d in range(3)]
        dmat = jnp.concatenate(blocks, axis=1)
        h = jnp.maximum(_dot(dmat, w1_ref[...]) + b1_ref[0:1, :], 0.0)
        h = jnp.maximum(_dot(h, w2_ref[...]) + b2_ref[0:1, :], 0.0)
        h = jnp.maximum(_dot(h, w3_ref[...]) + b3_ref[0:1, :], 0.0)
        fg = gg[:, 3:3 + C]
        h2 = _dot(_bf(h), r1_ref[...])
        fg2 = _dot(_bf(fg), r2_ref[...])
        f2 = jnp.sum((h2 * fg2).reshape(TM, K, JC), axis=1)
        o_ref[...] = _dot(_bf(f2), w2p_ref[...]) * (1.0 / K)

    full = lambda a: pl.BlockSpec(a.shape, lambda i: (0,) * a.ndim)
    return pl.pallas_call(
        body,
        grid=(rows // TM,),
        in_specs=[
            pl.BlockSpec((RK, D), lambda i: (i, 0)),
            pl.BlockSpec((TM, 3), lambda i: (i, 0)),
            full(cen), full(prm['l1w']), full(l1b),
            full(prm['l2w']), full(l2b),
            full(prm['l3w']), full(l3b), full(r1m), full(r2m), full(w2p),
        ],
        out_specs=pl.BlockSpec((TM, cout), lambda i: (i, 0)),
        out_shape=jax.ShapeDtypeStruct((rows, cout), jnp.float32),
        interpret=_INTERPRET,
    )(g, q, cen, prm['l1w'], l1b, prm['l2w'], l2b, prm['l3w'], l3b,
      r1m, r2m, w2p)


def _compute_fused(idx, queries, table, prm, K, C):
    B, M, _ = queries.shape
    N = table.shape[1]
    CC = 3 + C
    nc = _NC
    JC = nc * C
    TM = min(M, 256 if N <= 256 else 128,
             max(8, 1 << (3_000_000 // (K * JC)).bit_length() - 1))
    RK = TM * K
    cout = prm['weight'].shape[2]

    cen = jnp.pad(prm['centers'], ((0, 5), (0, 0)))
    l1b = jnp.broadcast_to(prm['l1b'][None, :], (8, 2 * nc))
    l2b = jnp.broadcast_to(prm['l2b'][None, :], (8, nc))
    l3b = jnp.broadcast_to(prm['l3b'][None, :], (8, nc))
    r1m = jnp.repeat(jnp.eye(nc, dtype=jnp.float32), C, axis=1)
    r2m = jnp.tile(jnp.eye(C, dtype=jnp.float32), (1, nc))
    w2p = jnp.swapaxes(prm['weight'], 0, 1).reshape(JC, cout)

    def body(idx_ref, q_ref, t_ref, cen_ref, w1_ref, b1_ref, w2_ref, b2_ref,
             w3_ref, b3_ref, r1_ref, r2_ref, w2p_ref, o_ref):
        b = pl.program_id(0)
        idxl = idx_ref[0] - b * N
        tb = t_ref[0]
        ii = jax.lax.broadcasted_iota(jnp.int32, (RK, N), 1)
        oh = jnp.where(ii == idxl, 1.0, 0.0)
        coords = jax.lax.dot_general(oh, tb[:, 0:3], (((1,), (0,)), ((), ())),
                                     precision=_HIGH,
                                     preferred_element_type=jnp.float32)
        feats = _dot(oh, tb[:, 3:CC])
        qq = q_ref[0]
        qrep = jnp.broadcast_to(qq[:, None, :], (TM, K, 3)).reshape(RK, 3)
        rel = coords - qrep
        blocks = [rel[:, d:d + 1] - cen_ref[d:d + 1, :] for d in range(3)]
        dmat = jnp.concatenate(blocks, axis=1)
        h = jnp.maximum(_dot(dmat, w1_ref[...]) + b1_ref[0:1, :], 0.0)
        h = jnp.maximum(_dot(h, w2_ref[...]) + b2_ref[0:1, :], 0.0)
        h = jnp.maximum(_dot(h, w3_ref[...]) + b3_ref[0:1, :], 0.0)
        h2 = _dot(_bf(h), r1_ref[...])
        fg2 = _dot(_bf(feats), r2_ref[...])
        f2 = jnp.sum((h2 * fg2).reshape(TM, K, JC), axis=1)
        o_ref[0] = _dot(_bf(f2), w2p_ref[...]) * (1.0 / K)

    full = lambda a: pl.BlockSpec(a.shape, lambda b, i: (0,) * a.ndim)
    return pl.pallas_call(
        body,
        grid=(B, M // TM),
        in_specs=[
            pl.BlockSpec((1, RK, 1), lambda b, i: (b, i, 0)),
            pl.BlockSpec((1, TM, 3), lambda b, i: (b, i, 0)),
            pl.BlockSpec((1, N, CC), lambda b, i: (b, 0, 0)),
            full(cen), full(prm['l1w']), full(l1b),
            full(prm['l2w']), full(l2b),
            full(prm['l3w']), full(l3b), full(r1m), full(r2m), full(w2p),
        ],
        out_specs=pl.BlockSpec((1, TM, cout), lambda b, i: (b, i, 0)),
        out_shape=jax.ShapeDtypeStruct((B, M, cout), jnp.float32),
        interpret=_INTERPRET,
    )(idx.reshape(B, M * K, 1), queries, table, cen, prm['l1w'], l1b,
      prm['l2w'], l2b, prm['l3w'], l3b, r1m, r2m, w2p)


def _bn_relu(x):
    B, M, C = x.shape
    x2 = x.reshape(B * M, C)

    def body(x_ref, o_ref):
        v = x_ref[...]
        mean = jnp.mean(v, axis=0, keepdims=True)
        xc = v - mean
        var = jnp.mean(xc * xc, axis=0, keepdims=True)
        o_ref[...] = jnp.maximum(xc / jnp.sqrt(var + 1e-5), 0.0)

    out = pl.pallas_call(
        body,
        out_shape=jax.ShapeDtypeStruct(x2.shape, jnp.float32),
        interpret=_INTERPRET,
    )(x2)
    return out.reshape(B, M, C)


def _classifier(x, w, b):
    B, M, C = x.shape
    x2 = x.reshape(B * M, C)
    cout = w.shape[1]
    b2 = jnp.broadcast_to(b[None, :], (8, cout))
    TM = min(B * M, 8192)

    def body(x_ref, w_ref, b_ref, o_ref):
        o_ref[...] = _dot(x_ref[...], w_ref[...]) + b_ref[0:1, :]

    out = pl.pallas_call(
        body,
        grid=((B * M) // TM,),
        in_specs=[
            pl.BlockSpec((TM, C), lambda i: (i, 0)),
            pl.BlockSpec(w.shape, lambda i: (0, 0)),
            pl.BlockSpec(b2.shape, lambda i: (0, 0)),
        ],
        out_specs=pl.BlockSpec((TM, cout), lambda i: (i, 0)),
        out_shape=jax.ShapeDtypeStruct((B * M, cout), jnp.float32),
        interpret=_INTERPRET,
    )(x2, w, b2)
    return out.reshape(B, M, cout)


def _ptconv(prm, feats, points, K, next_pts):
    B, N, C = feats.shape
    if isinstance(next_pts, int):
        if points.shape[1] != next_pts:
            stride = points.shape[1] // next_pts
            queries = points[:, ::stride, :][:, :next_pts, :]
        else:
            queries = points
    else:
        queries = next_pts
    M = queries.shape[1]

    idx = _knn(points, queries, K)
    if N <= 1024 and B * M * K * N <= 100_000_000:
        table3 = jnp.concatenate([points, feats], axis=2)
        out = _compute_fused(idx, queries, table3, prm, K, C)
        return out, queries
    D = _round_up(3 + C, 128)
    table = jnp.concatenate([points, feats], axis=2).reshape(B * N, 3 + C)
    if D > 3 + C:
        table = jnp.pad(table, ((0, 0), (0, D - (3 + C))))
    g = _sc_gather(table, idx.reshape(-1, 128))
    out = _compute(g, queries.reshape(B * M, 3), prm, K, C, D)
    return out.reshape(B, M, -1), queries


def kernel(x, input_pts, params):
    P = params
    x2, pts2 = _ptconv(P['cv2'], x, input_pts, 16, 1024)
    x2 = _bn_relu(x2)
    x3, pts3 = _ptconv(P['cv3'], x2, pts2, 16, 256)
    x3 = _bn_relu(x3)
    x4, pts4 = _ptconv(P['cv4'], x3, pts3, 8, 64)
    x4 = _bn_relu(x4)
    x5, pts5 = _ptconv(P['cv5'], x4, pts4, 8, 16)
    x5 = _bn_relu(x5)
    x6, pts6 = _ptconv(P['cv6'], x5, pts5, 4, 8)
    x6 = _bn_relu(x6)
    x5d, _ = _ptconv(P['cv5d'], x6, pts6, 4, pts5)
    x5d = _bn_relu(x5d)
    x5d = jnp.concatenate([x5d, x5], axis=2)
    x4d, _ = _ptconv(P['cv4d'], x5d, pts5, 4, pts4)
    x4d = _bn_relu(x4d)
    x4d = jnp.concatenate([x4d, x4], axis=2)
    x3d, _ = _ptconv(P['cv3d'], x4d, pts4, 4, pts3)
    x3d = _bn_relu(x3d)
    x3d = jnp.concatenate([x3d, x3], axis=2)
    x2d, _ = _ptconv(P['cv2d'], x3d, pts3, 8, pts2)
    x2d = _bn_relu(x2d)
    x2d = jnp.concatenate([x2d, x2], axis=2)
    x1d, _ = _ptconv(P['cv1d'], x2d, pts2, 8, input_pts)
    x1d = _bn_relu(x1d)
    return _classifier(x1d, P['fcout_w'], P['fcout_b'])


# speedup vs baseline: 1.0529x; 1.0280x over previous
import functools
import math

import jax
import jax.numpy as jnp
from jax import lax
from jax.experimental import pallas as pl
from jax.experimental.pallas import tpu as pltpu
from jax.experimental.pallas import tpu_sc as plsc

_INTERPRET = False
_NC = 16
_HIGH = jax.lax.Precision.HIGHEST


def _round_up(a, b):
    return (a + b - 1) // b * b


def _dot(a, b):
    return jax.lax.dot_general(a, b, (((1,), (0,)), ((), ())),
                               preferred_element_type=jnp.float32)


def _bf(x):
    return x.astype(jnp.bfloat16).astype(jnp.float32)


def _knn(points, queries, K):
    B, N, _ = points.shape
    M = queries.shape[1]
    TM = min(M, 512)
    ptT = jnp.swapaxes(points, 1, 2)

    def body(pt_ref, q_ref, o_ref):
        b = pl.program_id(0)
        pt = pt_ref[0]
        q = q_ref[0]
        qp = _dot(q, pt)
        psq = jnp.sum(pt * pt, axis=0, keepdims=True)
        qsq = jnp.sum(q * q, axis=1, keepdims=True)
        d2 = qsq - 2.0 * qp + psq
        iota = jax.lax.broadcasted_iota(jnp.int32, (TM, N), 1) + b * N
        big_i = jnp.int32(2**31 - 1)
        cols = []
        for _ in range(K):
            vmin = jnp.min(d2, axis=1, keepdims=True)
            eq = d2 == vmin
            sel = jnp.min(jnp.where(eq, iota, big_i), axis=1, keepdims=True)
            cols.append(sel)
            d2 = jnp.where(iota == sel, jnp.float32(jnp.inf), d2)
        o_ref[0] = jnp.concatenate(cols, axis=1)

    return pl.pallas_call(
        body,
        grid=(B, M // TM),
        in_specs=[
            pl.BlockSpec((1, 3, N), lambda b, i: (b, 0, 0)),
            pl.BlockSpec((1, TM, 3), lambda b, i: (b, i, 0)),
        ],
        out_specs=pl.BlockSpec((1, TM, K), lambda b, i: (b, i, 0)),
        out_shape=jax.ShapeDtypeStruct((B, M, K), jnp.int32),
        interpret=_INTERPRET,
    )(ptT, queries)


@functools.lru_cache(maxsize=None)
def _make_sc_gather(T, D, R):
    nchunks = R // 128
    NW = 32
    c_per_w = -(-nchunks // NW)
    mesh = plsc.VectorSubcoreMesh(core_axis_name="c", subcore_axis_name="s")

    @functools.partial(
        pl.kernel, mesh=mesh,
        out_type=jax.ShapeDtypeStruct((R, D), jnp.float32),
        scratch_types=[
            pltpu.VMEM((128,), jnp.int32),
            pltpu.VMEM((128, D), jnp.float32),
            pltpu.SemaphoreType.DMA,
        ],
    )
    def k(table_hbm, idx_hbm, out_hbm, idx_v, rows_v, sem):
        wid = lax.axis_index("s") * 2 + lax.axis_index("c")

        def body(j, carry):
            chunk = wid + NW * j

            @pl.when(chunk < nchunks)
            def _():
                pltpu.sync_copy(idx_hbm.at[chunk], idx_v)
                pltpu.async_copy(table_hbm.at[idx_v], rows_v, sem).wait()
                pltpu.sync_copy(rows_v, out_hbm.at[pl.ds(chunk * 128, 128)])
            return carry

        lax.fori_loop(0, c_per_w, body, 0)

    return k


def _sc_gather(table, idx2d):
    T, D = table.shape
    R = idx2d.shape[0] * 128
    return _make_sc_gather(T, D, R)(table, idx2d)


def _compute(g, q, prm, K, C, D):
    rows = q.shape[0]
    nc = _NC
    JC = nc * C
    TM = min(rows, max(8, 1 << (6_000_000 // (K * JC)).bit_length() - 1),
             256)
    RK = TM * K
    cout = prm['weight'].shape[2]

    cen = jnp.pad(prm['centers'], ((0, 5), (0, 0)))
    l1b = jnp.broadcast_to(prm['l1b'][None, :], (8, 2 * nc))
    l2b = jnp.broadcast_to(prm['l2b'][None, :], (8, nc))
    l3b = jnp.broadcast_to(prm['l3b'][None, :], (8, nc))
    r1m = jnp.repeat(jnp.eye(nc, dtype=jnp.float32), C, axis=1)
    r2m = jnp.tile(jnp.eye(C, dtype=jnp.float32), (1, nc))
    w2p = jnp.swapaxes(prm['weight'], 0, 1).reshape(JC, cout)

    def body(g_ref, q_ref, cen_ref, w1_ref, b1_ref, w2_ref, b2_ref,
             w3_ref, b3_ref, r1_ref, r2_ref, w2p_ref, o_ref):
        gg = g_ref[...]
        qq = q_ref[...]
        qrep = jnp.broadcast_to(qq[:, None, :], (TM, K, 3)).reshape(RK, 3)
        rel = gg[:, 0:3] - qrep
        blocks = [rel[:, d:d + 1] - cen_ref[d:d + 1, :] for d in range(3)]
        dmat = jnp.concatenate(blocks, axis=1)
        h = jnp.maximum(_dot(dmat, w1_ref[...]) + b1_ref[0:1, :], 0.0)
        h = jnp.maximum(_dot(h, w2_ref[...]) + b2_ref[0:1, :], 0.0)
        h = jnp.maximum(_dot(h, w3_ref[...]) + b3_ref[0:1, :], 0.0)
        fg = gg[:, 3:3 + C]
        h2 = _dot(_bf(h), r1_ref[...])
        fg2 = _dot(_bf(fg), r2_ref[...])
        f2 = jnp.sum((h2 * fg2).reshape(TM, K, JC), axis=1)
        o_ref[...] = _dot(_bf(f2), w2p_ref[...]) * (1.0 / K)

    full = lambda a: pl.BlockSpec(a.shape, lambda i: (0,) * a.ndim)
    return pl.pallas_call(
        body,
        grid=(rows // TM,),
        in_specs=[
            pl.BlockSpec((RK, D), lambda i: (i, 0)),
            pl.BlockSpec((TM, 3), lambda i: (i, 0)),
            full(cen), full(prm['l1w']), full(l1b),
            full(prm['l2w']), full(l2b),
            full(prm['l3w']), full(l3b), full(r1m), full(r2m), full(w2p),
        ],
        out_specs=pl.BlockSpec((TM, cout), lambda i: (i, 0)),
        out_shape=jax.ShapeDtypeStruct((rows, cout), jnp.float32),
        interpret=_INTERPRET,
    )(g, q, cen, prm['l1w'], l1b, prm['l2w'], l2b, prm['l3w'], l3b,
      r1m, r2m, w2p)


def _compute_fused(idx, queries, table, prm, K, C):
    B, M, _ = queries.shape
    N = table.shape[1]
    CC = 3 + C
    nc = _NC
    JC = nc * C
    TM = min(M, 256 if N <= 256 else 128,
             max(8, 1 << (6_000_000 // (K * JC)).bit_length() - 1))
    RK = TM * K
    cout = prm['weight'].shape[2]

    cen = jnp.pad(prm['centers'], ((0, 5), (0, 0)))
    l1b = jnp.broadcast_to(prm['l1b'][None, :], (8, 2 * nc))
    l2b = jnp.broadcast_to(prm['l2b'][None, :], (8, nc))
    l3b = jnp.broadcast_to(prm['l3b'][None, :], (8, nc))
    r1m = jnp.repeat(jnp.eye(nc, dtype=jnp.float32), C, axis=1)
    r2m = jnp.tile(jnp.eye(C, dtype=jnp.float32), (1, nc))
    w2p = jnp.swapaxes(prm['weight'], 0, 1).reshape(JC, cout)

    def body(idx_ref, q_ref, t_ref, cen_ref, w1_ref, b1_ref, w2_ref, b2_ref,
             w3_ref, b3_ref, r1_ref, r2_ref, w2p_ref, o_ref):
        b = pl.program_id(0)
        idxl = idx_ref[0] - b * N
        tb = t_ref[0]
        ii = jax.lax.broadcasted_iota(jnp.int32, (RK, N), 1)
        oh = jnp.where(ii == idxl, 1.0, 0.0)
        coords = jax.lax.dot_general(oh, tb[:, 0:3], (((1,), (0,)), ((), ())),
                                     precision=_HIGH,
                                     preferred_element_type=jnp.float32)
        feats = _dot(oh, tb[:, 3:CC])
        qq = q_ref[0]
        qrep = jnp.broadcast_to(qq[:, None, :], (TM, K, 3)).reshape(RK, 3)
        rel = coords - qrep
        blocks = [rel[:, d:d + 1] - cen_ref[d:d + 1, :] for d in range(3)]
        dmat = jnp.concatenate(blocks, axis=1)
        h = jnp.maximum(_dot(dmat, w1_ref[...]) + b1_ref[0:1, :], 0.0)
        h = jnp.maximum(_dot(h, w2_ref[...]) + b2_ref[0:1, :], 0.0)
        h = jnp.maximum(_dot(h, w3_ref[...]) + b3_ref[0:1, :], 0.0)
        h2 = _dot(_bf(h), r1_ref[...])
        fg2 = _dot(_bf(feats), r2_ref[...])
        f2 = jnp.sum((h2 * fg2).reshape(TM, K, JC), axis=1)
        o_ref[0] = _dot(_bf(f2), w2p_ref[...]) * (1.0 / K)

    full = lambda a: pl.BlockSpec(a.shape, lambda b, i: (0,) * a.ndim)
    return pl.pallas_call(
        body,
        grid=(B, M // TM),
        in_specs=[
            pl.BlockSpec((1, RK, 1), lambda b, i: (b, i, 0)),
            pl.BlockSpec((1, TM, 3), lambda b, i: (b, i, 0)),
            pl.BlockSpec((1, N, CC), lambda b, i: (b, 0, 0)),
            full(cen), full(prm['l1w']), full(l1b),
            full(prm['l2w']), full(l2b),
            full(prm['l3w']), full(l3b), full(r1m), full(r2m), full(w2p),
        ],
        out_specs=pl.BlockSpec((1, TM, cout), lambda b, i: (b, i, 0)),
        out_shape=jax.ShapeDtypeStruct((B, M, cout), jnp.float32),
        interpret=_INTERPRET,
    )(idx.reshape(B, M * K, 1), queries, table, cen, prm['l1w'], l1b,
      prm['l2w'], l2b, prm['l3w'], l3b, r1m, r2m, w2p)


def _bn_relu(x):
    B, M, C = x.shape
    x2 = x.reshape(B * M, C)

    def body(x_ref, o_ref):
        v = x_ref[...]
        mean = jnp.mean(v, axis=0, keepdims=True)
        xc = v - mean
        var = jnp.mean(xc * xc, axis=0, keepdims=True)
        o_ref[...] = jnp.maximum(xc / jnp.sqrt(var + 1e-5), 0.0)

    out = pl.pallas_call(
        body,
        out_shape=jax.ShapeDtypeStruct(x2.shape, jnp.float32),
        interpret=_INTERPRET,
    )(x2)
    return out.reshape(B, M, C)


def _classifier(x, w, b):
    B, M, C = x.shape
    x2 = x.reshape(B * M, C)
    cout = w.shape[1]
    b2 = jnp.broadcast_to(b[None, :], (8, cout))
    TM = min(B * M, 8192)

    def body(x_ref, w_ref, b_ref, o_ref):
        o_ref[...] = _dot(x_ref[...], w_ref[...]) + b_ref[0:1, :]

    out = pl.pallas_call(
        body,
        grid=((B * M) // TM,),
        in_specs=[
            pl.BlockSpec((TM, C), lambda i: (i, 0)),
            pl.BlockSpec(w.shape, lambda i: (0, 0)),
            pl.BlockSpec(b2.shape, lambda i: (0, 0)),
        ],
        out_specs=pl.BlockSpec((TM, cout), lambda i: (i, 0)),
        out_shape=jax.ShapeDtypeStruct((B * M, cout), jnp.float32),
        interpret=_INTERPRET,
    )(x2, w, b2)
    return out.reshape(B, M, cout)


def _ptconv(prm, feats, points, K, next_pts):
    B, N, C = feats.shape
    if isinstance(next_pts, int):
        if points.shape[1] != next_pts:
            stride = points.shape[1] // next_pts
            queries = points[:, ::stride, :][:, :next_pts, :]
        else:
            queries = points
    else:
        queries = next_pts
    M = queries.shape[1]

    idx = _knn(points, queries, K)
    if N <= 1024 and B * M * K * N <= 100_000_000:
        table3 = jnp.concatenate([points, feats], axis=2)
        out = _compute_fused(idx, queries, table3, prm, K, C)
        return out, queries
    D = _round_up(3 + C, 128)
    table = jnp.concatenate([points, feats], axis=2).reshape(B * N, 3 + C)
    if D > 3 + C:
        table = jnp.pad(table, ((0, 0), (0, D - (3 + C))))
    g = _sc_gather(table, idx.reshape(-1, 128))
    out = _compute(g, queries.reshape(B * M, 3), prm, K, C, D)
    return out.reshape(B, M, -1), queries


def kernel(x, input_pts, params):
    P = params
    x2, pts2 = _ptconv(P['cv2'], x, input_pts, 16, 1024)
    x2 = _bn_relu(x2)
    x3, pts3 = _ptconv(P['cv3'], x2, pts2, 16, 256)
    x3 = _bn_relu(x3)
    x4, pts4 = _ptconv(P['cv4'], x3, pts3, 8, 64)
    x4 = _bn_relu(x4)
    x5, pts5 = _ptconv(P['cv5'], x4, pts4, 8, 16)
    x5 = _bn_relu(x5)
    x6, pts6 = _ptconv(P['cv6'], x5, pts5, 4, 8)
    x6 = _bn_relu(x6)
    x5d, _ = _ptconv(P['cv5d'], x6, pts6, 4, pts5)
    x5d = _bn_relu(x5d)
    x5d = jnp.concatenate([x5d, x5], axis=2)
    x4d, _ = _ptconv(P['cv4d'], x5d, pts5, 4, pts4)
    x4d = _bn_relu(x4d)
    x4d = jnp.concatenate([x4d, x4], axis=2)
    x3d, _ = _ptconv(P['cv3d'], x4d, pts4, 4, pts3)
    x3d = _bn_relu(x3d)
    x3d = jnp.concatenate([x3d, x3], axis=2)
    x2d, _ = _ptconv(P['cv2d'], x3d, pts3, 8, pts2)
    x2d = _bn_relu(x2d)
    x2d = jnp.concatenate([x2d, x2], axis=2)
    x1d, _ = _ptconv(P['cv1d'], x2d, pts2, 8, input_pts)
    x1d = _bn_relu(x1d)
    return _classifier(x1d, P['fcout_w'], P['fcout_b'])
